# Initial kernel scaffold; baseline (speedup 1.0000x reference)
import jax
import jax.numpy as jnp
from jax.experimental import pallas as pl


def kernel(Z, centroids):
    raise NotImplementedError("write your pallas kernel here")


# speedup vs baseline: 6.6046x; 6.6046x over previous
import functools

import jax
import jax.numpy as jnp
from jax.experimental import pallas as pl

_BN = 512


def _kmeans_kernel(n_total, z_ref, c_ref, loss_ref, cl_ref):
    i = pl.program_id(0)
    z = z_ref[...]
    c = c_ref[...]
    g = jax.lax.dot_general(
        c, z, (((1,), (1,)), ((), ())),
        precision=jax.lax.Precision.HIGHEST,
        preferred_element_type=jnp.float32)
    cn = jnp.sum(c * c, axis=1, keepdims=True)
    s = cn - 2.0 * g
    m = jnp.min(s, axis=0, keepdims=True)
    k = s.shape[0]
    iota = jax.lax.broadcasted_iota(jnp.int32, s.shape, 0)
    idx = jnp.where(s == m, iota, k)
    cl_ref[...] = jnp.min(idx, axis=0, keepdims=True)[None]
    zn = jnp.sum(z * z, axis=1, keepdims=True)
    part = (jnp.sum(m) + jnp.sum(zn)).reshape(1, 1)

    @pl.when(i == 0)
    def _init():
        loss_ref[...] = part

    @pl.when(i > 0)
    def _acc():
        loss_ref[...] += part

    @pl.when(i == pl.num_programs(0) - 1)
    def _fin():
        loss_ref[...] = loss_ref[...] * (1.0 / n_total)


def kernel(Z, centroids):
    n, d = Z.shape
    kk = centroids.shape[0]
    nblk = n // _BN
    loss, cl = pl.pallas_call(
        functools.partial(_kmeans_kernel, n),
        grid=(nblk,),
        in_specs=[
            pl.BlockSpec((_BN, d), lambda i: (i, 0)),
            pl.BlockSpec((kk, d), lambda i: (0, 0)),
        ],
        out_specs=(
            pl.BlockSpec((1, 1), lambda i: (0, 0)),
            pl.BlockSpec((1, 1, _BN), lambda i: (i, 0, 0)),
        ),
        out_shape=(
            jax.ShapeDtypeStruct((1, 1), jnp.float32),
            jax.ShapeDtypeStruct((nblk, 1, _BN), jnp.int32),
        ),
    )(Z, centroids)
    return (loss[0, 0], cl.reshape(n))


# speedup vs baseline: 11.8136x; 1.7887x over previous
import functools

import jax
import jax.numpy as jnp
from jax.experimental import pallas as pl


def _kmeans_kernel(n_total, zt_ref, ct_ref, loss_ref, cl_ref):
    zt = zt_ref[...]
    ct = ct_ref[...]
    cn = jnp.sum(ct * ct, axis=0, keepdims=True)
    ca = jnp.concatenate([ct * -2.0, cn], axis=0)
    za = jnp.concatenate(
        [zt, jnp.ones((1, zt.shape[1]), jnp.float32)], axis=0)
    s = jax.lax.dot_general(
        ca, za, (((0,), (0,)), ((), ())),
        precision=jax.lax.Precision.HIGHEST,
        preferred_element_type=jnp.float32)
    m = jnp.min(s, axis=0, keepdims=True)
    k = s.shape[0]
    iota = jax.lax.broadcasted_iota(jnp.int32, s.shape, 0)
    idx = jnp.min(jnp.where(s == m, iota, k), axis=0, keepdims=True)
    cl_ref[...] = idx
    zn = jnp.sum(zt * zt, axis=0, keepdims=True)
    loss_ref[...] = (jnp.sum(m + zn) * (1.0 / n_total)).reshape(1, 1)


def kernel(Z, centroids):
    n, d = Z.shape
    kk = centroids.shape[0]
    zt = jnp.swapaxes(Z, 0, 1)
    ct = jnp.swapaxes(centroids, 0, 1)
    loss, cl = pl.pallas_call(
        functools.partial(_kmeans_kernel, n),
        out_shape=(
            jax.ShapeDtypeStruct((1, 1), jnp.float32),
            jax.ShapeDtypeStruct((1, n), jnp.int32),
        ),
    )(zt, ct)
    return (loss[0, 0], cl.reshape(n))


# speedup vs baseline: 12.3602x; 1.0463x over previous
import functools

import jax
import jax.numpy as jnp
from jax.experimental import pallas as pl


def _kmeans_kernel(n_total, zt_ref, ct_ref, loss_ref, cl_ref):
    zt = zt_ref[...]
    ct = ct_ref[...]
    cn = jnp.sum(ct * ct, axis=0, keepdims=True)
    ca = jnp.concatenate([ct * -2.0, cn], axis=0)
    za = jnp.concatenate(
        [zt, jnp.ones((1, zt.shape[1]), jnp.float32)], axis=0)
    s = jax.lax.dot_general(
        ca, za, (((0,), (0,)), ((), ())),
        precision=jax.lax.Precision.HIGHEST,
        preferred_element_type=jnp.float32)
    m = jnp.min(s, axis=0, keepdims=True)
    cl_ref[...] = jnp.argmin(s, axis=0)[None]
    zn = jnp.sum(zt * zt, axis=0, keepdims=True)
    loss_ref[...] = (jnp.sum(m + zn) * (1.0 / n_total)).reshape(1, 1)


def kernel(Z, centroids):
    n, d = Z.shape
    zt = jnp.swapaxes(Z, 0, 1)
    ct = jnp.swapaxes(centroids, 0, 1)
    loss, cl = pl.pallas_call(
        functools.partial(_kmeans_kernel, n),
        out_shape=(
            jax.ShapeDtypeStruct((1, 1), jnp.float32),
            jax.ShapeDtypeStruct((1, n), jnp.int32),
        ),
    )(zt, ct)
    return (loss[0, 0], cl.reshape(n))
